# initial kernel scaffold (unmeasured)
import functools

import jax
import jax.numpy as jnp
from jax import lax
from jax.experimental import pallas as pl
from jax.experimental.pallas import tpu as pltpu

N_DEV = 4


def kernel(x, w_mat):
    k_tot, k_shard = x.shape
    _, n = w_mat.shape
    m_per = k_tot // N_DEV

    def body(x_ref, w_ref, out_ref, comm_ref, send_sems, recv_sems):
        me = lax.axis_index("i")

        barrier_sem = pltpu.get_barrier_semaphore()
        for d in range(1, N_DEV):
            peer = lax.rem(me + d, N_DEV)
            pl.semaphore_signal(
                barrier_sem, inc=1,
                device_id=(peer,), device_id_type=pl.DeviceIdType.MESH,
            )
        pl.semaphore_wait(barrier_sem, N_DEV - 1)

        rdmas = []
        for d in range(1, N_DEV):
            target = lax.rem(me + d, N_DEV)
            rdma = pltpu.make_async_remote_copy(
                src_ref=x_ref.at[pl.ds(target * m_per, m_per), :],
                dst_ref=comm_ref.at[d - 1],
                send_sem=send_sems.at[d - 1],
                recv_sem=recv_sems.at[d - 1],
                device_id=(target,),
                device_id_type=pl.DeviceIdType.MESH,
            )
            rdma.start()
            rdmas.append(rdma)

        local = x_ref[pl.ds(me * m_per, m_per), :]
        w_blk = w_ref[pl.ds(me * k_shard, k_shard), :]
        acc = jnp.dot(local, w_blk, preferred_element_type=jnp.float32)

        for d in range(1, N_DEV):
            rdmas[d - 1].wait_recv()
            src = lax.rem(me - d + N_DEV, N_DEV)
            w_blk = w_ref[pl.ds(src * k_shard, k_shard), :]
            acc = acc + jnp.dot(
                comm_ref[d - 1], w_blk, preferred_element_type=jnp.float32
            )

        out_ref[:, :] = jnp.maximum(acc, 0.0)

        for d in range(1, N_DEV):
            rdmas[d - 1].wait_send()

        @functools.partial(pl.run_scoped, sem2=pltpu.SemaphoreType.REGULAR)
        def _(sem2):
            for d in range(1, N_DEV):
                peer = lax.rem(me + d, N_DEV)
                pl.semaphore_signal(
                    sem2, inc=1,
                    device_id=(peer,), device_id_type=pl.DeviceIdType.MESH,
                )
            pl.semaphore_wait(sem2, N_DEV - 1)

    return pl.pallas_call(
        body,
        out_shape=jax.ShapeDtypeStruct((m_per, n), jnp.float32),
        in_specs=[
            pl.BlockSpec(memory_space=pltpu.VMEM),
            pl.BlockSpec(memory_space=pltpu.VMEM),
        ],
        out_specs=pl.BlockSpec(memory_space=pltpu.VMEM),
        scratch_shapes=[
            pltpu.VMEM((N_DEV - 1, m_per, k_shard), jnp.float32),
            pltpu.SemaphoreType.DMA((N_DEV - 1,)),
            pltpu.SemaphoreType.DMA((N_DEV - 1,)),
        ],
        compiler_params=pltpu.CompilerParams(collective_id=0),
    )(x, w_mat)


# baseline (device time: 121848 ns/iter reference)
import functools

import jax
import jax.numpy as jnp
from jax import lax
from jax.experimental import pallas as pl
from jax.experimental.pallas import tpu as pltpu

N_DEV = 4
N_TILES = 2


def kernel(x, w_mat):
    k_tot, k_shard = x.shape
    _, n = w_mat.shape
    m_per = k_tot // N_DEV
    n_half = n // N_TILES

    d_order = [0, 1, 3, 2]
    chunks = [(d, nh) for d in d_order for nh in range(N_TILES)]

    def body(x_ref, w_ref, out_ref, comm_ref, w_buf, send_sems, recv_sems,
             w_sems):
        me = lax.axis_index("i")

        barrier_sem = pltpu.get_barrier_semaphore()
        for d in range(1, N_DEV):
            peer = lax.rem(me + d, N_DEV)
            pl.semaphore_signal(
                barrier_sem, inc=1,
                device_id=(peer,), device_id_type=pl.DeviceIdType.MESH,
            )
        pl.semaphore_wait(barrier_sem, N_DEV - 1)

        rdmas = []
        for d in range(1, N_DEV):
            target = lax.rem(me + d, N_DEV)
            rdma = pltpu.make_async_remote_copy(
                src_ref=x_ref.at[pl.ds(target * m_per, m_per), :],
                dst_ref=comm_ref.at[d - 1],
                send_sem=send_sems.at[d - 1],
                recv_sem=recv_sems.at[d - 1],
                device_id=(target,),
                device_id_type=pl.DeviceIdType.MESH,
            )
            rdma.start()
            rdmas.append(rdma)

        def k_block(d):
            return lax.rem(me - d + N_DEV, N_DEV)

        def start_w_load(c):
            d, nh = chunks[c]
            copy = pltpu.make_async_copy(
                w_ref.at[pl.ds(k_block(d) * k_shard, k_shard),
                         pl.ds(nh * n_half, n_half)],
                w_buf.at[c % 2],
                w_sems.at[c % 2],
            )
            copy.start()
            return copy

        def wait_w_load(c):
            d, nh = chunks[c]
            pltpu.make_async_copy(
                w_ref.at[pl.ds(k_block(d) * k_shard, k_shard),
                         pl.ds(nh * n_half, n_half)],
                w_buf.at[c % 2],
                w_sems.at[c % 2],
            ).wait()

        start_w_load(0)
        start_w_load(1)

        for c, (d, nh) in enumerate(chunks):
            if nh == 0 and d != 0:
                rdmas[d - 1].wait_recv()
            wait_w_load(c)
            if d == 0:
                lhs = x_ref[pl.ds(me * m_per, m_per), :]
            else:
                lhs = comm_ref[d - 1]
            partial = jnp.dot(
                lhs, w_buf[c % 2], preferred_element_type=jnp.float32
            )
            nsl = pl.ds(nh * n_half, n_half)
            if c < N_TILES:
                out_ref[:, nsl] = partial
            else:
                out_ref[:, nsl] = out_ref[:, nsl] + partial
            if c + 2 < len(chunks):
                start_w_load(c + 2)

        out_ref[:, :] = jnp.maximum(out_ref[:, :], 0.0)

        for d in range(1, N_DEV):
            rdmas[d - 1].wait_send()

        @functools.partial(pl.run_scoped, sem2=pltpu.SemaphoreType.REGULAR)
        def _(sem2):
            for d in range(1, N_DEV):
                peer = lax.rem(me + d, N_DEV)
                pl.semaphore_signal(
                    sem2, inc=1,
                    device_id=(peer,), device_id_type=pl.DeviceIdType.MESH,
                )
            pl.semaphore_wait(sem2, N_DEV - 1)

    return pl.pallas_call(
        body,
        out_shape=jax.ShapeDtypeStruct((m_per, n), jnp.float32),
        in_specs=[
            pl.BlockSpec(memory_space=pltpu.VMEM),
            pl.BlockSpec(memory_space=pl.ANY),
        ],
        out_specs=pl.BlockSpec(memory_space=pltpu.VMEM),
        scratch_shapes=[
            pltpu.VMEM((N_DEV - 1, m_per, k_shard), jnp.float32),
            pltpu.VMEM((2, k_shard, n_half), jnp.float32),
            pltpu.SemaphoreType.DMA((N_DEV - 1,)),
            pltpu.SemaphoreType.DMA((N_DEV - 1,)),
            pltpu.SemaphoreType.DMA((2,)),
        ],
        compiler_params=pltpu.CompilerParams(
            collective_id=0,
            vmem_limit_bytes=60 * 1024 * 1024,
        ),
    )(x, w_mat)


# device time: 77885 ns/iter; 1.5645x vs baseline; 1.5645x over previous
import functools

import jax
import jax.numpy as jnp
from jax import lax
from jax.experimental import pallas as pl
from jax.experimental.pallas import tpu as pltpu

N_DEV = 4
N_TILES = 2


def kernel(x, w_mat):
    k_tot, k_shard = x.shape
    _, n = w_mat.shape
    m_per = k_tot // N_DEV
    n_half = n // N_TILES

    d_order = [0, 1, 3, 2]
    chunks = [(d, nh) for d in d_order for nh in range(N_TILES)]

    def body(x_ref, w_ref, out_ref, x_bf, comm_ref, w_buf, w_bf,
             send_sems, recv_sems, w_sems):
        me = lax.axis_index("i")

        barrier_sem = pltpu.get_barrier_semaphore()
        for d in range(1, N_DEV):
            peer = lax.rem(me + d, N_DEV)
            pl.semaphore_signal(
                barrier_sem, inc=1,
                device_id=(peer,), device_id_type=pl.DeviceIdType.MESH,
            )
        pl.semaphore_wait(barrier_sem, N_DEV - 1)

        x_bf[:, :] = x_ref[:, :].astype(jnp.bfloat16)

        rdmas = []
        for d in range(1, N_DEV):
            target = lax.rem(me + d, N_DEV)
            rdma = pltpu.make_async_remote_copy(
                src_ref=x_bf.at[pl.ds(target * m_per, m_per), :],
                dst_ref=comm_ref.at[d - 1],
                send_sem=send_sems.at[d - 1],
                recv_sem=recv_sems.at[d - 1],
                device_id=(target,),
                device_id_type=pl.DeviceIdType.MESH,
            )
            rdma.start()
            rdmas.append(rdma)

        def k_block(d):
            return lax.rem(me - d + N_DEV, N_DEV)

        def start_w_load(c):
            d, nh = chunks[c]
            pltpu.make_async_copy(
                w_ref.at[pl.ds(k_block(d) * k_shard, k_shard),
                         pl.ds(nh * n_half, n_half)],
                w_buf.at[c % 2],
                w_sems.at[c % 2],
            ).start()

        def wait_w_load(c):
            d, nh = chunks[c]
            pltpu.make_async_copy(
                w_ref.at[pl.ds(k_block(d) * k_shard, k_shard),
                         pl.ds(nh * n_half, n_half)],
                w_buf.at[c % 2],
                w_sems.at[c % 2],
            ).wait()

        start_w_load(0)
        start_w_load(1)

        for c, (d, nh) in enumerate(chunks):
            if nh == 0 and d != 0:
                rdmas[d - 1].wait_recv()
            wait_w_load(c)
            w_bf[c % 2] = w_buf[c % 2].astype(jnp.bfloat16)
            if d == 0:
                lhs = x_bf[pl.ds(me * m_per, m_per), :]
            else:
                lhs = comm_ref[d - 1]
            partial = jnp.dot(
                lhs, w_bf[c % 2], preferred_element_type=jnp.float32
            )
            nsl = pl.ds(nh * n_half, n_half)
            if c < N_TILES:
                out_ref[:, nsl] = partial
            else:
                out_ref[:, nsl] = out_ref[:, nsl] + partial
            if c + 2 < len(chunks):
                start_w_load(c + 2)

        out_ref[:, :] = jnp.maximum(out_ref[:, :], 0.0)

        for d in range(1, N_DEV):
            rdmas[d - 1].wait_send()

        @functools.partial(pl.run_scoped, sem2=pltpu.SemaphoreType.REGULAR)
        def _(sem2):
            for d in range(1, N_DEV):
                peer = lax.rem(me + d, N_DEV)
                pl.semaphore_signal(
                    sem2, inc=1,
                    device_id=(peer,), device_id_type=pl.DeviceIdType.MESH,
                )
            pl.semaphore_wait(sem2, N_DEV - 1)

    return pl.pallas_call(
        body,
        out_shape=jax.ShapeDtypeStruct((m_per, n), jnp.float32),
        in_specs=[
            pl.BlockSpec(memory_space=pltpu.VMEM),
            pl.BlockSpec(memory_space=pl.ANY),
        ],
        out_specs=pl.BlockSpec(memory_space=pltpu.VMEM),
        scratch_shapes=[
            pltpu.VMEM((k_tot, k_shard), jnp.bfloat16),
            pltpu.VMEM((N_DEV - 1, m_per, k_shard), jnp.bfloat16),
            pltpu.VMEM((2, k_shard, n_half), jnp.float32),
            pltpu.VMEM((2, k_shard, n_half), jnp.bfloat16),
            pltpu.SemaphoreType.DMA((N_DEV - 1,)),
            pltpu.SemaphoreType.DMA((N_DEV - 1,)),
            pltpu.SemaphoreType.DMA((2,)),
        ],
        compiler_params=pltpu.CompilerParams(
            collective_id=0,
            vmem_limit_bytes=60 * 1024 * 1024,
        ),
    )(x, w_mat)


# device time: 77530 ns/iter; 1.5716x vs baseline; 1.0046x over previous
import functools

import jax
import jax.numpy as jnp
from jax import lax
from jax.experimental import pallas as pl
from jax.experimental.pallas import tpu as pltpu

N_DEV = 4
N_TILES = 2


def kernel(x, w_mat):
    k_tot, k_shard = x.shape
    _, n = w_mat.shape
    m_per = k_tot // N_DEV
    n_half = n // N_TILES

    send_order = [2, 1, 3, 0]
    d_order = [0, 1, 3, 2]
    chunks = [(d, nh) for d in d_order for nh in range(N_TILES)]

    def body(x_ref, w_ref, out_ref, x_land, x_bf, comm_ref, w_buf, w_bf,
             send_sems, recv_sems, x_sems, w_sems):
        me = lax.axis_index("i")

        barrier_sem = pltpu.get_barrier_semaphore()
        for d in range(1, N_DEV):
            peer = lax.rem(me + d, N_DEV)
            pl.semaphore_signal(
                barrier_sem, inc=1,
                device_id=(peer,), device_id_type=pl.DeviceIdType.MESH,
            )
        pl.semaphore_wait(barrier_sem, N_DEV - 1)

        def x_copy(b):
            d = send_order[b]
            blk = lax.rem(me + d, N_DEV)
            return pltpu.make_async_copy(
                x_ref.at[pl.ds(blk * m_per, m_per), :],
                x_land.at[b % 2],
                x_sems.at[b % 2],
            )

        x_copy(0).start()
        x_copy(1).start()
        rdmas = {}
        for b, d in enumerate(send_order):
            x_copy(b).wait()
            x_bf[b] = x_land[b % 2].astype(jnp.bfloat16)
            if b + 2 < N_DEV:
                x_copy(b + 2).start()
            if d != 0:
                target = lax.rem(me + d, N_DEV)
                rdma = pltpu.make_async_remote_copy(
                    src_ref=x_bf.at[b],
                    dst_ref=comm_ref.at[d - 1],
                    send_sem=send_sems.at[d - 1],
                    recv_sem=recv_sems.at[d - 1],
                    device_id=(target,),
                    device_id_type=pl.DeviceIdType.MESH,
                )
                rdma.start()
                rdmas[d] = rdma

        def k_block(d):
            return lax.rem(me - d + N_DEV, N_DEV)

        def w_copy(c):
            d, nh = chunks[c]
            return pltpu.make_async_copy(
                w_ref.at[pl.ds(k_block(d) * k_shard, k_shard),
                         pl.ds(nh * n_half, n_half)],
                w_buf.at[c % 2],
                w_sems.at[c % 2],
            )

        w_copy(0).start()
        w_copy(1).start()

        local_slot = send_order.index(0)
        for c, (d, nh) in enumerate(chunks):
            if nh == 0 and d != 0:
                rdmas[d].wait_recv()
            w_copy(c).wait()
            w_bf[c % 2] = w_buf[c % 2].astype(jnp.bfloat16)
            if d == 0:
                lhs = x_bf[local_slot]
            else:
                lhs = comm_ref[d - 1]
            partial = jnp.dot(
                lhs, w_bf[c % 2], preferred_element_type=jnp.float32
            )
            nsl = pl.ds(nh * n_half, n_half)
            if c < N_TILES:
                out_ref[:, nsl] = partial
            else:
                out_ref[:, nsl] = out_ref[:, nsl] + partial
            if c + 2 < len(chunks):
                w_copy(c + 2).start()

        out_ref[:, :] = jnp.maximum(out_ref[:, :], 0.0)

        for d in range(1, N_DEV):
            rdmas[d].wait_send()

        @functools.partial(pl.run_scoped, sem2=pltpu.SemaphoreType.REGULAR)
        def _(sem2):
            for d in range(1, N_DEV):
                peer = lax.rem(me + d, N_DEV)
                pl.semaphore_signal(
                    sem2, inc=1,
                    device_id=(peer,), device_id_type=pl.DeviceIdType.MESH,
                )
            pl.semaphore_wait(sem2, N_DEV - 1)

    return pl.pallas_call(
        body,
        out_shape=jax.ShapeDtypeStruct((m_per, n), jnp.float32),
        in_specs=[
            pl.BlockSpec(memory_space=pl.ANY),
            pl.BlockSpec(memory_space=pl.ANY),
        ],
        out_specs=pl.BlockSpec(memory_space=pltpu.VMEM),
        scratch_shapes=[
            pltpu.VMEM((2, m_per, k_shard), jnp.float32),
            pltpu.VMEM((N_DEV, m_per, k_shard), jnp.bfloat16),
            pltpu.VMEM((N_DEV - 1, m_per, k_shard), jnp.bfloat16),
            pltpu.VMEM((2, k_shard, n_half), jnp.float32),
            pltpu.VMEM((2, k_shard, n_half), jnp.bfloat16),
            pltpu.SemaphoreType.DMA((N_DEV - 1,)),
            pltpu.SemaphoreType.DMA((N_DEV - 1,)),
            pltpu.SemaphoreType.DMA((2,)),
            pltpu.SemaphoreType.DMA((2,)),
        ],
        compiler_params=pltpu.CompilerParams(
            collective_id=0,
            vmem_limit_bytes=60 * 1024 * 1024,
        ),
    )(x, w_mat)
